# baseline (speedup 1.0000x reference)
import functools

import jax
import jax.numpy as jnp
from jax import lax
from jax.experimental import pallas as pl
from jax.experimental.pallas import tpu as pltpu
from jax.experimental.pallas import tpu_sc as plsc

KNN = 32
DIL = 2
K64 = KNN * DIL


_RV = 128
_NL = 64


def _bf16_round(x):
    r = lax.bitcast_convert_type(x, jnp.int32)
    r = (r + 0x7FFF + ((r >> 16) & 1)) & ~0xFFFF
    return lax.bitcast_convert_type(r, jnp.float32)


def _cmpx(k0, p0, k1, p1, asc):
    less = k0 < k1
    sel = less == asc
    return (jnp.where(sel, k0, k1), jnp.where(sel, p0, p1),
            jnp.where(sel, k1, k0), jnp.where(sel, p1, p0))


def _sort64_ax1(key, pay, asc_list):
    r = key.shape[2]
    for k in (2, 4, 8, 16, 32, 64):
        j = k // 2
        while j >= 1:
            g = _NL // (2 * j)
            ks = key.reshape(2, g, 2, j, r)
            ps = pay.reshape(2, g, 2, j, r)
            if k == 64:
                asc = asc_list
            else:
                giota = lax.broadcasted_iota(jnp.int32, (1, g, 1, 1), 1)
                asc = (((giota * (2 * j)) & k) == 0) == asc_list
            k0, p0, k1, p1 = _cmpx(ks[:, :, 0], ps[:, :, 0],
                                   ks[:, :, 1], ps[:, :, 1], asc)
            key = jnp.stack([k0, k1], axis=2).reshape(2, _NL, r)
            pay = jnp.stack([p0, p1], axis=2).reshape(2, _NL, r)
            j //= 2
    return key, pay


def _merge64_ax0(key, pay, asc):
    r = key.shape[1]
    for j in (32, 16, 8, 4, 2, 1):
        g = _NL // (2 * j)
        ks = key.reshape(g, 2, j, r)
        ps = pay.reshape(g, 2, j, r)
        k0, p0, k1, p1 = _cmpx(ks[:, 0], ps[:, 0], ks[:, 1], ps[:, 1], asc)
        key = jnp.stack([k0, k1], axis=1).reshape(_NL, r)
        pay = jnp.stack([p0, p1], axis=1).reshape(_NL, r)
    return key, pay


def _halver_merge(key2, pay2, asc):
    kx, ky = key2[0], key2[1]
    px, py = pay2[0], pay2[1]
    less = kx < ky
    key = jnp.where(less, kx, ky)
    pay = jnp.where(less, px, py)
    return _merge64_ax0(key, pay, asc)


def _sort_body(pts_row_ref, pts_all_ref, key_ref, pay_ref):
    t = pl.program_id(2)
    pr = pts_row_ref[0]
    pr16 = _bf16_round(pr)
    pa = pts_all_ref[0]
    pa16 = _bf16_round(pa)
    inner = (pa16[:, 0:1] * pr16[0:1, :] + pa16[:, 1:2] * pr16[1:2, :]
             + pa16[:, 2:3] * pr16[2:3, :])
    sq_r = (pr[0:1, :] * pr[0:1, :] + pr[1:2, :] * pr[1:2, :]
            + pr[2:3, :] * pr[2:3, :])
    sq_c = (pa[:, 0:1] * pa[:, 0:1] + pa[:, 1:2] * pa[:, 1:2]
            + pa[:, 2:3] * pa[:, 2:3])
    dist = (sq_r - 2.0 * inner) + sq_c

    key = dist.reshape(2, _NL, _RV)
    pay = (lax.broadcasted_iota(jnp.int32, (2, _NL, _RV), 0) * _NL
           + lax.broadcasted_iota(jnp.int32, (2, _NL, _RV), 1)) + t * 128

    asc_list = lax.broadcasted_iota(jnp.int32, (2, 1, 1, 1), 0) == 0
    key, pay = _sort64_ax1(key, pay, asc_list)
    key, pay = _halver_merge(key, pay, (t & 1) == 0)
    key_ref[0, 0, 0] = key
    pay_ref[0, 0, 0] = pay


def _merge_body(final, key_ref, pay_ref, ko_ref, po_ref):
    t = pl.program_id(2)
    asc_out = ((t & 1) == 0) | final
    key, pay = _halver_merge(key_ref[0, 0], pay_ref[0, 0], asc_out)
    if final:
        po_ref[0, 0] = jnp.transpose(pay, (1, 0))
    else:
        ko_ref[0, 0, 0] = key
        po_ref[0, 0, 0] = pay


def _topk64(points):
    b, _, n = points.shape
    nrb = n // _RV
    nt = n // 128
    pts_t = jnp.transpose(points, (0, 2, 1))

    key, pay = pl.pallas_call(
        _sort_body,
        grid=(b, nrb, nt),
        in_specs=[
            pl.BlockSpec((1, 3, _RV), lambda i, j, t: (i, 0, j)),
            pl.BlockSpec((1, 128, 3), lambda i, j, t: (i, t, 0)),
        ],
        out_specs=[
            pl.BlockSpec((1, 1, 1, _NL, _RV), lambda i, j, t: (i, j, t, 0, 0)),
            pl.BlockSpec((1, 1, 1, _NL, _RV), lambda i, j, t: (i, j, t, 0, 0)),
        ],
        out_shape=[
            jax.ShapeDtypeStruct((b, nrb, nt, _NL, _RV), jnp.float32),
            jax.ShapeDtypeStruct((b, nrb, nt, _NL, _RV), jnp.int32),
        ],
    )(points, pts_t)

    l = nt
    while l > 1:
        l2 = l // 2
        final = l2 == 1
        out_specs = [
            pl.BlockSpec((1, 1, 1, _NL, _RV), lambda i, j, t: (i, j, t, 0, 0)),
            (pl.BlockSpec((1, 1, _RV, _NL), lambda i, j, t: (i, j, 0, 0))
             if final else
             pl.BlockSpec((1, 1, 1, _NL, _RV), lambda i, j, t: (i, j, t, 0, 0))),
        ]
        out_shape = [
            jax.ShapeDtypeStruct((b, nrb, l2, _NL, _RV), jnp.float32),
            (jax.ShapeDtypeStruct((b, nrb, _RV, K64), jnp.int32)
             if final else
             jax.ShapeDtypeStruct((b, nrb, l2, _NL, _RV), jnp.int32)),
        ]
        key, pay = pl.pallas_call(
            functools.partial(_merge_body, final),
            grid=(b, nrb, l2),
            in_specs=[
                pl.BlockSpec((1, 1, 2, _NL, _RV),
                             lambda i, j, t: (i, j, t, 0, 0)),
                pl.BlockSpec((1, 1, 2, _NL, _RV),
                             lambda i, j, t: (i, j, t, 0, 0)),
            ],
            out_specs=out_specs,
            out_shape=out_shape,
        )(key, pay)
        l = l2

    return pay.reshape(b, n, K64)


def _node_mlp_body(f_ref, w0_ref, b0_ref, w1_ref, b1_ref, w2_ref, b2_ref,
                   m_ref):
    f = f_ref[0]
    a = lax.dot_general(f, w0_ref[...], (((0,), (1,)), ((), ())),
                        preferred_element_type=jnp.float32)
    a = jnp.maximum(a + b0_ref[...], 0.0)
    a = lax.dot_general(a, w1_ref[...], (((1,), (1,)), ((), ())),
                        preferred_element_type=jnp.float32)
    a = jnp.maximum(a + b1_ref[...], 0.0)
    a = lax.dot_general(a, w2_ref[...], (((1,), (1,)), ((), ())),
                        preferred_element_type=jnp.float32)
    m_ref[0] = a + b2_ref[...]


def _node_mlp(features, w0, b0, w1, b1, w2, b2):
    b, c, n = features.shape
    co = w2.shape[0]
    full = lambda a: pl.BlockSpec(a.shape, lambda i: (0,) * a.ndim)
    args = (w0, b0.reshape(1, -1), w1, b1.reshape(1, -1), w2,
            b2.reshape(1, -1))
    return pl.pallas_call(
        _node_mlp_body,
        grid=(b,),
        in_specs=[pl.BlockSpec((1, c, n), lambda i: (i, 0, 0))] +
                 [full(a) for a in args],
        out_specs=pl.BlockSpec((1, n, co), lambda i: (i, 0, 0)),
        out_shape=jax.ShapeDtypeStruct((b, n, co), jnp.float32),
    )(features, *args)


_NC = 2
_NS = 16
_NW = _NC * _NS
_CHUNK = 4


def _gather_max_sc(idx1, idx2, t1, t2):
    bn, c = t1.shape
    pw = bn // _NW
    nchunks = pw // _CHUNK
    rows_per_chunk = _CHUNK * KNN
    idx_rows_pw = pw * KNN // 128

    mesh = plsc.VectorSubcoreMesh(core_axis_name="c", subcore_axis_name="s")

    @functools.partial(
        pl.kernel,
        mesh=mesh,
        out_type=[jax.ShapeDtypeStruct((bn, c), jnp.float32),
                  jax.ShapeDtypeStruct((bn, c), jnp.float32)],
        scratch_types=[
            pltpu.VMEM((idx_rows_pw, 128), jnp.int32),
            pltpu.VMEM((rows_per_chunk, c), jnp.float32),
            pltpu.VMEM((rows_per_chunk, c), jnp.float32),
            pltpu.VMEM((pw, c), jnp.float32),
            pltpu.SemaphoreType.DMA,
            pltpu.SemaphoreType.DMA,
        ],
    )
    def kern(idx1_hbm, idx2_hbm, t1_hbm, t2_hbm, l1_hbm, l2_hbm,
             idx_v, rows_a, rows_b, out_v, sem_a, sem_b):
        w = lax.axis_index("s") * _NC + lax.axis_index("c")

        def compute(rows_v, cbase):
            for nloc in range(_CHUNK):
                node = cbase * _CHUNK + nloc
                for j in range(c // 16):
                    acc = rows_v[nloc * KNN, pl.ds(j * 16, 16)]
                    for k in range(1, KNN):
                        acc = jnp.maximum(
                            acc, rows_v[nloc * KNN + k, pl.ds(j * 16, 16)])
                    out_v[node, pl.ds(j * 16, 16)] = acc

        for idx_hbm, t_hbm, l_hbm in ((idx1_hbm, t1_hbm, l1_hbm),
                                      (idx2_hbm, t2_hbm, l2_hbm)):
            pltpu.sync_copy(idx_hbm.at[pl.ds(w * idx_rows_pw, idx_rows_pw)],
                            idx_v)
            pltpu.async_copy(t_hbm.at[idx_v.at[0]], rows_a, sem_a)

            def pair_body(p, _, t_hbm=t_hbm):
                c0 = p * 2
                pltpu.async_copy(t_hbm.at[idx_v.at[c0 + 1]], rows_b, sem_b)
                pltpu.make_async_copy(t_hbm.at[idx_v.at[c0]],
                                      rows_a, sem_a).wait()
                compute(rows_a, c0)

                @pl.when(p < nchunks // 2 - 1)
                def _():
                    pltpu.async_copy(t_hbm.at[idx_v.at[c0 + 2]], rows_a, sem_a)

                pltpu.make_async_copy(t_hbm.at[idx_v.at[c0 + 1]],
                                      rows_b, sem_b).wait()
                compute(rows_b, c0 + 1)
                return 0

            lax.fori_loop(0, nchunks // 2, pair_body, 0)
            pltpu.sync_copy(out_v, l_hbm.at[pl.ds(w * pw, pw)])

    return kern(idx1, idx2, t1, t2)


_NB = 1024


def _final_mlp_body(l1_ref, l2_ref, w0a_ref, w0b_ref, b0_ref, w1_ref, b1_ref,
                    w2_ref, b2_ref, out_ref):
    z = (lax.dot_general(l1_ref[0], w0a_ref[...], (((1,), (1,)), ((), ())),
                         preferred_element_type=jnp.float32) +
         lax.dot_general(l2_ref[0], w0b_ref[...], (((1,), (1,)), ((), ())),
                         preferred_element_type=jnp.float32))
    z = jnp.maximum(z + b0_ref[...], 0.0)
    z = lax.dot_general(z, w1_ref[...], (((1,), (1,)), ((), ())),
                        preferred_element_type=jnp.float32)
    z = jnp.maximum(z + b1_ref[...], 0.0)
    out = lax.dot_general(w2_ref[...], z, (((1,), (1,)), ((), ())),
                          preferred_element_type=jnp.float32)
    out_ref[0] = out + b2_ref[...]


def _final_mlp(l1, l2, w0, b0, w1, b1, w2, b2):
    b, n, c = l1.shape
    c3 = w2.shape[0]
    w0a = w0[:, :c]
    w0b = w0[:, c:]
    full = lambda a: pl.BlockSpec(a.shape, lambda i, j: (0,) * a.ndim)
    args = (w0a, w0b, b0.reshape(1, -1), w1, b1.reshape(1, -1), w2,
            b2.reshape(-1, 1))
    return pl.pallas_call(
        _final_mlp_body,
        grid=(b, n // _NB),
        in_specs=[pl.BlockSpec((1, _NB, c), lambda i, j: (i, j, 0)),
                  pl.BlockSpec((1, _NB, c), lambda i, j: (i, j, 0))] +
                 [full(a) for a in args],
        out_specs=pl.BlockSpec((1, c3, _NB), lambda i, j: (i, 0, j)),
        out_shape=jax.ShapeDtypeStruct((b, c3, n), jnp.float32),
    )(l1, l2, *args)


def kernel(points, features, m1_w0, m1_b0, m1_w1, m1_b1, m1_w2, m1_b2,
           m2_w0, m2_b0, m2_w1, m2_b1, m2_w2, m2_b2,
           mm_w0, mm_b0, mm_w1, mm_b1, mm_w2, mm_b2):
    b, c, n = features.shape
    bn = b * n

    idx64 = _topk64(points)

    m1 = _node_mlp(features, m1_w0, m1_b0, m1_w1, m1_b1, m1_w2, m1_b2)
    m2 = _node_mlp(features, m2_w0, m2_b0, m2_w1, m2_b1, m2_w2, m2_b2)

    offs = (jnp.arange(b, dtype=jnp.int32) * n)[:, None, None]
    idx1 = (idx64[:, :, :KNN] + offs).reshape(bn * KNN // 128, 128)
    idx2 = (idx64[:, :, ::DIL] + offs).reshape(bn * KNN // 128, 128)

    l1, l2 = _gather_max_sc(idx1, idx2,
                            m1.reshape(bn, c), m2.reshape(bn, c))

    return _final_mlp(l1.reshape(b, n, c), l2.reshape(b, n, c),
                      mm_w0, mm_b0, mm_w1, mm_b1, mm_w2, mm_b2)


# speedup vs baseline: 1.1763x; 1.1763x over previous
import functools

import jax
import jax.numpy as jnp
from jax import lax
from jax.experimental import pallas as pl
from jax.experimental.pallas import tpu as pltpu
from jax.experimental.pallas import tpu_sc as plsc

KNN = 32
DIL = 2
K64 = KNN * DIL


_RV = 128
_NL = 64


def _bf16_round(x):
    r = lax.bitcast_convert_type(x, jnp.int32)
    r = (r + 0x7FFF + ((r >> 16) & 1)) & ~0xFFFF
    return lax.bitcast_convert_type(r, jnp.float32)


def _cmpx(k0, p0, k1, p1, asc):
    less = k0 < k1
    sel = less == asc
    return (jnp.where(sel, k0, k1), jnp.where(sel, p0, p1),
            jnp.where(sel, k1, k0), jnp.where(sel, p1, p0))


def _sort64_ax1(key, pay, asc_list):
    r = key.shape[2]
    for k in (2, 4, 8, 16, 32, 64):
        j = k // 2
        while j >= 1:
            g = _NL // (2 * j)
            ks = key.reshape(2, g, 2, j, r)
            ps = pay.reshape(2, g, 2, j, r)
            if k == 64:
                asc = asc_list
            else:
                giota = lax.broadcasted_iota(jnp.int32, (1, g, 1, 1), 1)
                asc = (((giota * (2 * j)) & k) == 0) == asc_list
            k0, p0, k1, p1 = _cmpx(ks[:, :, 0], ps[:, :, 0],
                                   ks[:, :, 1], ps[:, :, 1], asc)
            key = jnp.stack([k0, k1], axis=2).reshape(2, _NL, r)
            pay = jnp.stack([p0, p1], axis=2).reshape(2, _NL, r)
            j //= 2
    return key, pay


def _merge64_ax0(key, pay, asc):
    r = key.shape[1]
    for j in (32, 16, 8, 4, 2, 1):
        g = _NL // (2 * j)
        ks = key.reshape(g, 2, j, r)
        ps = pay.reshape(g, 2, j, r)
        k0, p0, k1, p1 = _cmpx(ks[:, 0], ps[:, 0], ks[:, 1], ps[:, 1], asc)
        key = jnp.stack([k0, k1], axis=1).reshape(_NL, r)
        pay = jnp.stack([p0, p1], axis=1).reshape(_NL, r)
    return key, pay


def _halver_merge(key2, pay2, asc):
    kx, ky = key2[0], key2[1]
    px, py = pay2[0], pay2[1]
    less = kx < ky
    key = jnp.where(less, kx, ky)
    pay = jnp.where(less, px, py)
    return _merge64_ax0(key, pay, asc)


def _topk_body(pts_row_ref, pts_all_ref, idx_ref, key_s, pay_s):
    nt = pts_all_ref.shape[1] // 128
    pr = pts_row_ref[0]
    pr16 = _bf16_round(pr)
    sq_r = (pr[0:1, :] * pr[0:1, :] + pr[1:2, :] * pr[1:2, :]
            + pr[2:3, :] * pr[2:3, :])

    def sort_tile(t, _):
        pa = pts_all_ref[0, pl.ds(t * 128, 128), :]
        pa16 = _bf16_round(pa)
        inner = (pa16[:, 0:1] * pr16[0:1, :] + pa16[:, 1:2] * pr16[1:2, :]
                 + pa16[:, 2:3] * pr16[2:3, :])
        sq_c = (pa[:, 0:1] * pa[:, 0:1] + pa[:, 1:2] * pa[:, 1:2]
                + pa[:, 2:3] * pa[:, 2:3])
        dist = (sq_r - 2.0 * inner) + sq_c

        key = dist.reshape(2, _NL, _RV)
        pay = (lax.broadcasted_iota(jnp.int32, (2, _NL, _RV), 0) * _NL
               + lax.broadcasted_iota(jnp.int32, (2, _NL, _RV), 1)) + t * 128
        asc_list = lax.broadcasted_iota(jnp.int32, (2, 1, 1, 1), 0) == 0
        key, pay = _sort64_ax1(key, pay, asc_list)
        key, pay = _halver_merge(key, pay, (t & 1) == 0)
        key_s[pl.ds(t * _NL, _NL), :] = key
        pay_s[pl.ds(t * _NL, _NL), :] = pay
        return 0

    lax.fori_loop(0, nt, sort_tile, 0)

    l = nt
    while l > 2:
        l2 = l // 2

        def round_body(t, _):
            k2 = key_s[pl.ds(t * 128, 128), :].reshape(2, _NL, _RV)
            p2 = pay_s[pl.ds(t * 128, 128), :].reshape(2, _NL, _RV)
            key, pay = _halver_merge(k2, p2, (t & 1) == 0)
            key_s[pl.ds(t * _NL, _NL), :] = key
            pay_s[pl.ds(t * _NL, _NL), :] = pay
            return 0

        lax.fori_loop(0, l2, round_body, 0)
        l = l2

    k2 = key_s[0:128, :].reshape(2, _NL, _RV)
    p2 = pay_s[0:128, :].reshape(2, _NL, _RV)
    _, pay = _halver_merge(k2, p2, True)
    idx_ref[0] = jnp.transpose(pay, (1, 0))


def _topk64(points):
    b, _, n = points.shape
    nrb = n // _RV
    pts_t = jnp.transpose(points, (0, 2, 1))
    return pl.pallas_call(
        _topk_body,
        grid=(b, nrb),
        in_specs=[
            pl.BlockSpec((1, 3, _RV), lambda i, j: (i, 0, j)),
            pl.BlockSpec((1, n, 3), lambda i, j: (i, 0, 0)),
        ],
        out_specs=pl.BlockSpec((1, _RV, K64), lambda i, j: (i, j, 0)),
        out_shape=jax.ShapeDtypeStruct((b, n, K64), jnp.int32),
        scratch_shapes=[
            pltpu.VMEM((n // 2, _RV), jnp.float32),
            pltpu.VMEM((n // 2, _RV), jnp.int32),
        ],
    )(points, pts_t)


def _node_mlp_body(f_ref, w0_ref, b0_ref, w1_ref, b1_ref, w2_ref, b2_ref,
                   m_ref):
    f = f_ref[0]
    a = lax.dot_general(f, w0_ref[...], (((0,), (1,)), ((), ())),
                        preferred_element_type=jnp.float32)
    a = jnp.maximum(a + b0_ref[...], 0.0)
    a = lax.dot_general(a, w1_ref[...], (((1,), (1,)), ((), ())),
                        preferred_element_type=jnp.float32)
    a = jnp.maximum(a + b1_ref[...], 0.0)
    a = lax.dot_general(a, w2_ref[...], (((1,), (1,)), ((), ())),
                        preferred_element_type=jnp.float32)
    m_ref[0] = a + b2_ref[...]


def _node_mlp(features, w0, b0, w1, b1, w2, b2):
    b, c, n = features.shape
    co = w2.shape[0]
    full = lambda a: pl.BlockSpec(a.shape, lambda i: (0,) * a.ndim)
    args = (w0, b0.reshape(1, -1), w1, b1.reshape(1, -1), w2,
            b2.reshape(1, -1))
    return pl.pallas_call(
        _node_mlp_body,
        grid=(b,),
        in_specs=[pl.BlockSpec((1, c, n), lambda i: (i, 0, 0))] +
                 [full(a) for a in args],
        out_specs=pl.BlockSpec((1, n, co), lambda i: (i, 0, 0)),
        out_shape=jax.ShapeDtypeStruct((b, n, co), jnp.float32),
    )(features, *args)


_NC = 2
_NS = 16
_NW = _NC * _NS
_CHUNK = 4


def _gather_max_sc(idx1, idx2, t1, t2):
    bn, c = t1.shape
    pw = bn // _NW
    nchunks = pw // _CHUNK
    rows_per_chunk = _CHUNK * KNN
    idx_rows_pw = pw * KNN // 128

    mesh = plsc.VectorSubcoreMesh(core_axis_name="c", subcore_axis_name="s")

    @functools.partial(
        pl.kernel,
        mesh=mesh,
        out_type=[jax.ShapeDtypeStruct((bn, c), jnp.float32),
                  jax.ShapeDtypeStruct((bn, c), jnp.float32)],
        scratch_types=[
            pltpu.VMEM((idx_rows_pw, 128), jnp.int32),
            pltpu.VMEM((rows_per_chunk, c), jnp.float32),
            pltpu.VMEM((rows_per_chunk, c), jnp.float32),
            pltpu.VMEM((pw, c), jnp.float32),
            pltpu.SemaphoreType.DMA,
            pltpu.SemaphoreType.DMA,
        ],
    )
    def kern(idx1_hbm, idx2_hbm, t1_hbm, t2_hbm, l1_hbm, l2_hbm,
             idx_v, rows_a, rows_b, out_v, sem_a, sem_b):
        w = lax.axis_index("s") * _NC + lax.axis_index("c")

        def compute(rows_v, cbase):
            for nloc in range(_CHUNK):
                node = cbase * _CHUNK + nloc
                for j in range(c // 16):
                    acc = rows_v[nloc * KNN, pl.ds(j * 16, 16)]
                    for k in range(1, KNN):
                        acc = jnp.maximum(
                            acc, rows_v[nloc * KNN + k, pl.ds(j * 16, 16)])
                    out_v[node, pl.ds(j * 16, 16)] = acc

        for idx_hbm, t_hbm, l_hbm in ((idx1_hbm, t1_hbm, l1_hbm),
                                      (idx2_hbm, t2_hbm, l2_hbm)):
            pltpu.sync_copy(idx_hbm.at[pl.ds(w * idx_rows_pw, idx_rows_pw)],
                            idx_v)
            pltpu.async_copy(t_hbm.at[idx_v.at[0]], rows_a, sem_a)

            def pair_body(p, _, t_hbm=t_hbm):
                c0 = p * 2
                pltpu.async_copy(t_hbm.at[idx_v.at[c0 + 1]], rows_b, sem_b)
                pltpu.make_async_copy(t_hbm.at[idx_v.at[c0]],
                                      rows_a, sem_a).wait()
                compute(rows_a, c0)

                @pl.when(p < nchunks // 2 - 1)
                def _():
                    pltpu.async_copy(t_hbm.at[idx_v.at[c0 + 2]], rows_a, sem_a)

                pltpu.make_async_copy(t_hbm.at[idx_v.at[c0 + 1]],
                                      rows_b, sem_b).wait()
                compute(rows_b, c0 + 1)
                return 0

            lax.fori_loop(0, nchunks // 2, pair_body, 0)
            pltpu.sync_copy(out_v, l_hbm.at[pl.ds(w * pw, pw)])

    return kern(idx1, idx2, t1, t2)


_NB = 1024


def _final_mlp_body(l1_ref, l2_ref, w0a_ref, w0b_ref, b0_ref, w1_ref, b1_ref,
                    w2_ref, b2_ref, out_ref):
    z = (lax.dot_general(l1_ref[0], w0a_ref[...], (((1,), (1,)), ((), ())),
                         preferred_element_type=jnp.float32) +
         lax.dot_general(l2_ref[0], w0b_ref[...], (((1,), (1,)), ((), ())),
                         preferred_element_type=jnp.float32))
    z = jnp.maximum(z + b0_ref[...], 0.0)
    z = lax.dot_general(z, w1_ref[...], (((1,), (1,)), ((), ())),
                        preferred_element_type=jnp.float32)
    z = jnp.maximum(z + b1_ref[...], 0.0)
    out = lax.dot_general(w2_ref[...], z, (((1,), (1,)), ((), ())),
                          preferred_element_type=jnp.float32)
    out_ref[0] = out + b2_ref[...]


def _final_mlp(l1, l2, w0, b0, w1, b1, w2, b2):
    b, n, c = l1.shape
    c3 = w2.shape[0]
    w0a = w0[:, :c]
    w0b = w0[:, c:]
    full = lambda a: pl.BlockSpec(a.shape, lambda i, j: (0,) * a.ndim)
    args = (w0a, w0b, b0.reshape(1, -1), w1, b1.reshape(1, -1), w2,
            b2.reshape(-1, 1))
    return pl.pallas_call(
        _final_mlp_body,
        grid=(b, n // _NB),
        in_specs=[pl.BlockSpec((1, _NB, c), lambda i, j: (i, j, 0)),
                  pl.BlockSpec((1, _NB, c), lambda i, j: (i, j, 0))] +
                 [full(a) for a in args],
        out_specs=pl.BlockSpec((1, c3, _NB), lambda i, j: (i, 0, j)),
        out_shape=jax.ShapeDtypeStruct((b, c3, n), jnp.float32),
    )(l1, l2, *args)


def kernel(points, features, m1_w0, m1_b0, m1_w1, m1_b1, m1_w2, m1_b2,
           m2_w0, m2_b0, m2_w1, m2_b1, m2_w2, m2_b2,
           mm_w0, mm_b0, mm_w1, mm_b1, mm_w2, mm_b2):
    b, c, n = features.shape
    bn = b * n

    idx64 = _topk64(points)

    m1 = _node_mlp(features, m1_w0, m1_b0, m1_w1, m1_b1, m1_w2, m1_b2)
    m2 = _node_mlp(features, m2_w0, m2_b0, m2_w1, m2_b1, m2_w2, m2_b2)

    offs = (jnp.arange(b, dtype=jnp.int32) * n)[:, None, None]
    idx1 = (idx64[:, :, :KNN] + offs).reshape(bn * KNN // 128, 128)
    idx2 = (idx64[:, :, ::DIL] + offs).reshape(bn * KNN // 128, 128)

    l1, l2 = _gather_max_sc(idx1, idx2,
                            m1.reshape(bn, c), m2.reshape(bn, c))

    return _final_mlp(l1.reshape(b, n, c), l2.reshape(b, n, c),
                      mm_w0, mm_b0, mm_w1, mm_b1, mm_w2, mm_b2)


# speedup vs baseline: 1.5782x; 1.3417x over previous
import functools

import jax
import jax.numpy as jnp
from jax import lax
from jax.experimental import pallas as pl
from jax.experimental.pallas import tpu as pltpu
from jax.experimental.pallas import tpu_sc as plsc

KNN = 32
DIL = 2
K64 = KNN * DIL


_RV = 128
_NL = 64


def _bf16_round(x):
    r = lax.bitcast_convert_type(x, jnp.int32)
    r = (r + 0x7FFF + ((r >> 16) & 1)) & ~0xFFFF
    return lax.bitcast_convert_type(r, jnp.float32)


def _neg_bits(kb):
    return kb ^ (-2147483648)


def _sort64_signed(kb, pay):
    l, r = kb.shape[1], kb.shape[2]
    io = lax.broadcasted_iota(jnp.int32, (_NL, 1, 1), 0)
    prev = None
    for k2 in (2, 4, 8, 16, 32, 64):
        pat = (io & k2) << (31 - k2.bit_length() + 1)
        kb = kb ^ pat if prev is None else kb ^ (prev ^ pat)
        prev = pat
        key = lax.bitcast_convert_type(kb, jnp.float32)
        j = k2 // 2
        while j >= 1:
            g = _NL // (2 * j)
            ks = key.reshape(g, 2, j, l, r)
            ps = pay.reshape(g, 2, j, l, r)
            a, b2 = ks[:, 0], ks[:, 1]
            pa_, pb_ = ps[:, 0], ps[:, 1]
            less = a < b2
            lo = jnp.minimum(a, b2)
            hi = jnp.maximum(a, b2)
            plo = jnp.where(less, pa_, pb_)
            phi = jnp.where(less, pb_, pa_)
            key = jnp.stack([lo, hi], axis=1).reshape(_NL, l, r)
            pay = jnp.stack([plo, phi], axis=1).reshape(_NL, l, r)
            j //= 2
        kb = lax.bitcast_convert_type(key, jnp.int32)
    return kb, pay


def _merge_round_signed(kb, pay):
    l, r = kb.shape[1], kb.shape[2]
    l2 = l // 2
    ks = kb.reshape(_NL, l2, 2, r)
    ps = pay.reshape(_NL, l2, 2, r)
    x = lax.bitcast_convert_type(ks[:, :, 0], jnp.float32)
    y = lax.bitcast_convert_type(_neg_bits(ks[:, :, 1]), jnp.float32)
    px, py = ps[:, :, 0], ps[:, :, 1]
    less = x < y
    key = jnp.minimum(x, y)
    pay = jnp.where(less, px, py)
    lio = lax.broadcasted_iota(jnp.int32, (1, l2, 1), 1)
    key = lax.bitcast_convert_type(
        lax.bitcast_convert_type(key, jnp.int32) ^ ((lio & 1) << 31),
        jnp.float32)
    for j in (32, 16, 8, 4, 2, 1):
        g = _NL // (2 * j)
        ksj = key.reshape(g, 2, j, l2, r)
        psj = pay.reshape(g, 2, j, l2, r)
        a, b2 = ksj[:, 0], ksj[:, 1]
        pa_, pb_ = psj[:, 0], psj[:, 1]
        lessj = a < b2
        lo = jnp.minimum(a, b2)
        hi = jnp.maximum(a, b2)
        plo = jnp.where(lessj, pa_, pb_)
        phi = jnp.where(lessj, pb_, pa_)
        key = jnp.stack([lo, hi], axis=1).reshape(_NL, l2, r)
        pay = jnp.stack([plo, phi], axis=1).reshape(_NL, l2, r)
    return lax.bitcast_convert_type(key, jnp.int32), pay


def _topk_body(pts_row_ref, pts_all_ref, idx_ref):
    n = pts_all_ref.shape[1]
    nl2 = n // _NL
    pr = pts_row_ref[0]
    pr16 = _bf16_round(pr)
    pa = pts_all_ref[0]
    pa16 = _bf16_round(pa)
    inner = (pa16[:, 0:1] * pr16[0:1, :] + pa16[:, 1:2] * pr16[1:2, :]
             + pa16[:, 2:3] * pr16[2:3, :])
    sq_r = (pr[0:1, :] * pr[0:1, :] + pr[1:2, :] * pr[1:2, :]
            + pr[2:3, :] * pr[2:3, :])
    sq_c = (pa[:, 0:1] * pa[:, 0:1] + pa[:, 1:2] * pa[:, 1:2]
            + pa[:, 2:3] * pa[:, 2:3])
    dist = (sq_r - 2.0 * inner) + sq_c

    kb = lax.bitcast_convert_type(dist.reshape(_NL, nl2, _RV), jnp.int32)
    pay = (lax.broadcasted_iota(jnp.int32, (_NL, nl2, _RV), 0) * nl2
           + lax.broadcasted_iota(jnp.int32, (_NL, nl2, _RV), 1))
    lio = lax.broadcasted_iota(jnp.int32, (1, nl2, 1), 1)
    kb = kb ^ ((lio & 1) << 31)

    kb, pay = _sort64_signed(kb, pay)
    l = nl2
    while l > 1:
        kb, pay = _merge_round_signed(kb, pay)
        l //= 2

    idx_ref[0] = jnp.transpose(pay.reshape(_NL, _RV), (1, 0))


def _topk64(points):
    b, _, n = points.shape
    nrb = n // _RV
    pts_t = jnp.transpose(points, (0, 2, 1))
    return pl.pallas_call(
        _topk_body,
        grid=(b, nrb),
        in_specs=[
            pl.BlockSpec((1, 3, _RV), lambda i, j: (i, 0, j)),
            pl.BlockSpec((1, n, 3), lambda i, j: (i, 0, 0)),
        ],
        out_specs=pl.BlockSpec((1, _RV, K64), lambda i, j: (i, j, 0)),
        out_shape=jax.ShapeDtypeStruct((b, n, K64), jnp.int32),
    )(points, pts_t)


def _node_mlp_body(f_ref, w0_ref, b0_ref, w1_ref, b1_ref, w2_ref, b2_ref,
                   m_ref):
    f = f_ref[0]
    a = lax.dot_general(f, w0_ref[...], (((0,), (1,)), ((), ())),
                        preferred_element_type=jnp.float32)
    a = jnp.maximum(a + b0_ref[...], 0.0)
    a = lax.dot_general(a, w1_ref[...], (((1,), (1,)), ((), ())),
                        preferred_element_type=jnp.float32)
    a = jnp.maximum(a + b1_ref[...], 0.0)
    a = lax.dot_general(a, w2_ref[...], (((1,), (1,)), ((), ())),
                        preferred_element_type=jnp.float32)
    m_ref[0] = a + b2_ref[...]


def _node_mlp(features, w0, b0, w1, b1, w2, b2):
    b, c, n = features.shape
    co = w2.shape[0]
    full = lambda a: pl.BlockSpec(a.shape, lambda i: (0,) * a.ndim)
    args = (w0, b0.reshape(1, -1), w1, b1.reshape(1, -1), w2,
            b2.reshape(1, -1))
    return pl.pallas_call(
        _node_mlp_body,
        grid=(b,),
        in_specs=[pl.BlockSpec((1, c, n), lambda i: (i, 0, 0))] +
                 [full(a) for a in args],
        out_specs=pl.BlockSpec((1, n, co), lambda i: (i, 0, 0)),
        out_shape=jax.ShapeDtypeStruct((b, n, co), jnp.float32),
    )(features, *args)


_NC = 2
_NS = 16
_NW = _NC * _NS
_CHUNK = 4


def _gather_max_sc(idx1, idx2, t1, t2):
    bn, c = t1.shape
    pw = bn // _NW
    nchunks = pw // _CHUNK
    rows_per_chunk = _CHUNK * KNN
    idx_rows_pw = pw * KNN // 128

    mesh = plsc.VectorSubcoreMesh(core_axis_name="c", subcore_axis_name="s")

    @functools.partial(
        pl.kernel,
        mesh=mesh,
        out_type=[jax.ShapeDtypeStruct((bn, c), jnp.float32),
                  jax.ShapeDtypeStruct((bn, c), jnp.float32)],
        scratch_types=[
            pltpu.VMEM((idx_rows_pw, 128), jnp.int32),
            pltpu.VMEM((rows_per_chunk, c), jnp.float32),
            pltpu.VMEM((rows_per_chunk, c), jnp.float32),
            pltpu.VMEM((pw, c), jnp.float32),
            pltpu.SemaphoreType.DMA,
            pltpu.SemaphoreType.DMA,
        ],
    )
    def kern(idx1_hbm, idx2_hbm, t1_hbm, t2_hbm, l1_hbm, l2_hbm,
             idx_v, rows_a, rows_b, out_v, sem_a, sem_b):
        w = lax.axis_index("s") * _NC + lax.axis_index("c")

        def compute(rows_v, cbase):
            for nloc in range(_CHUNK):
                node = cbase * _CHUNK + nloc
                for j in range(c // 16):
                    acc = rows_v[nloc * KNN, pl.ds(j * 16, 16)]
                    for k in range(1, KNN):
                        acc = jnp.maximum(
                            acc, rows_v[nloc * KNN + k, pl.ds(j * 16, 16)])
                    out_v[node, pl.ds(j * 16, 16)] = acc

        for idx_hbm, t_hbm, l_hbm in ((idx1_hbm, t1_hbm, l1_hbm),
                                      (idx2_hbm, t2_hbm, l2_hbm)):
            pltpu.sync_copy(idx_hbm.at[pl.ds(w * idx_rows_pw, idx_rows_pw)],
                            idx_v)
            pltpu.async_copy(t_hbm.at[idx_v.at[0]], rows_a, sem_a)

            def pair_body(p, _, t_hbm=t_hbm):
                c0 = p * 2
                pltpu.async_copy(t_hbm.at[idx_v.at[c0 + 1]], rows_b, sem_b)
                pltpu.make_async_copy(t_hbm.at[idx_v.at[c0]],
                                      rows_a, sem_a).wait()
                compute(rows_a, c0)

                @pl.when(p < nchunks // 2 - 1)
                def _():
                    pltpu.async_copy(t_hbm.at[idx_v.at[c0 + 2]], rows_a, sem_a)

                pltpu.make_async_copy(t_hbm.at[idx_v.at[c0 + 1]],
                                      rows_b, sem_b).wait()
                compute(rows_b, c0 + 1)
                return 0

            lax.fori_loop(0, nchunks // 2, pair_body, 0)
            pltpu.sync_copy(out_v, l_hbm.at[pl.ds(w * pw, pw)])

    return kern(idx1, idx2, t1, t2)


_NB = 1024


def _final_mlp_body(l1_ref, l2_ref, w0a_ref, w0b_ref, b0_ref, w1_ref, b1_ref,
                    w2_ref, b2_ref, out_ref):
    z = (lax.dot_general(l1_ref[0], w0a_ref[...], (((1,), (1,)), ((), ())),
                         preferred_element_type=jnp.float32) +
         lax.dot_general(l2_ref[0], w0b_ref[...], (((1,), (1,)), ((), ())),
                         preferred_element_type=jnp.float32))
    z = jnp.maximum(z + b0_ref[...], 0.0)
    z = lax.dot_general(z, w1_ref[...], (((1,), (1,)), ((), ())),
                        preferred_element_type=jnp.float32)
    z = jnp.maximum(z + b1_ref[...], 0.0)
    out = lax.dot_general(w2_ref[...], z, (((1,), (1,)), ((), ())),
                          preferred_element_type=jnp.float32)
    out_ref[0] = out + b2_ref[...]


def _final_mlp(l1, l2, w0, b0, w1, b1, w2, b2):
    b, n, c = l1.shape
    c3 = w2.shape[0]
    w0a = w0[:, :c]
    w0b = w0[:, c:]
    full = lambda a: pl.BlockSpec(a.shape, lambda i, j: (0,) * a.ndim)
    args = (w0a, w0b, b0.reshape(1, -1), w1, b1.reshape(1, -1), w2,
            b2.reshape(-1, 1))
    return pl.pallas_call(
        _final_mlp_body,
        grid=(b, n // _NB),
        in_specs=[pl.BlockSpec((1, _NB, c), lambda i, j: (i, j, 0)),
                  pl.BlockSpec((1, _NB, c), lambda i, j: (i, j, 0))] +
                 [full(a) for a in args],
        out_specs=pl.BlockSpec((1, c3, _NB), lambda i, j: (i, 0, j)),
        out_shape=jax.ShapeDtypeStruct((b, c3, n), jnp.float32),
    )(l1, l2, *args)


def kernel(points, features, m1_w0, m1_b0, m1_w1, m1_b1, m1_w2, m1_b2,
           m2_w0, m2_b0, m2_w1, m2_b1, m2_w2, m2_b2,
           mm_w0, mm_b0, mm_w1, mm_b1, mm_w2, mm_b2):
    b, c, n = features.shape
    bn = b * n

    idx64 = _topk64(points)

    m1 = _node_mlp(features, m1_w0, m1_b0, m1_w1, m1_b1, m1_w2, m1_b2)
    m2 = _node_mlp(features, m2_w0, m2_b0, m2_w1, m2_b1, m2_w2, m2_b2)

    offs = (jnp.arange(b, dtype=jnp.int32) * n)[:, None, None]
    idx1 = (idx64[:, :, :KNN] + offs).reshape(bn * KNN // 128, 128)
    idx2 = (idx64[:, :, ::DIL] + offs).reshape(bn * KNN // 128, 128)

    l1, l2 = _gather_max_sc(idx1, idx2,
                            m1.reshape(bn, c), m2.reshape(bn, c))

    return _final_mlp(l1.reshape(b, n, c), l2.reshape(b, n, c),
                      mm_w0, mm_b0, mm_w1, mm_b1, mm_w2, mm_b2)


# speedup vs baseline: 1.6077x; 1.0187x over previous
import functools

import jax
import jax.numpy as jnp
from jax import lax
from jax.experimental import pallas as pl
from jax.experimental.pallas import tpu as pltpu
from jax.experimental.pallas import tpu_sc as plsc

KNN = 32
DIL = 2
K64 = KNN * DIL


_RV = 128
_NL = 64


def _bf16_round(x):
    r = lax.bitcast_convert_type(x, jnp.int32)
    r = (r + 0x7FFF + ((r >> 16) & 1)) & ~0xFFFF
    return lax.bitcast_convert_type(r, jnp.float32)


def _neg_bits(kb):
    return kb ^ (-2147483648)


def _sort64_signed(kb, pay):
    l, r = kb.shape[1], kb.shape[2]
    io = lax.broadcasted_iota(jnp.int32, (_NL, 1, 1), 0)
    prev = None
    for k2 in (2, 4, 8, 16, 32, 64):
        pat = (io & k2) << (31 - k2.bit_length() + 1)
        kb = kb ^ pat if prev is None else kb ^ (prev ^ pat)
        prev = pat
        key = lax.bitcast_convert_type(kb, jnp.float32)
        j = k2 // 2
        while j >= 1:
            g = _NL // (2 * j)
            ks = key.reshape(g, 2, j, l, r)
            ps = pay.reshape(g, 2, j, l, r)
            a, b2 = ks[:, 0], ks[:, 1]
            pa_, pb_ = ps[:, 0], ps[:, 1]
            less = a < b2
            lo = jnp.minimum(a, b2)
            hi = jnp.maximum(a, b2)
            plo = jnp.where(less, pa_, pb_)
            phi = jnp.where(less, pb_, pa_)
            key = jnp.stack([lo, hi], axis=1).reshape(_NL, l, r)
            pay = jnp.stack([plo, phi], axis=1).reshape(_NL, l, r)
            j //= 2
        kb = lax.bitcast_convert_type(key, jnp.int32)
    return kb, pay


def _merge_round_signed(kb, pay):
    l, r = kb.shape[1], kb.shape[2]
    l2 = l // 2
    ks = kb.reshape(_NL, l2, 2, r)
    ps = pay.reshape(_NL, l2, 2, r)
    x = lax.bitcast_convert_type(ks[:, :, 0], jnp.float32)
    y = lax.bitcast_convert_type(_neg_bits(ks[:, :, 1]), jnp.float32)
    px, py = ps[:, :, 0], ps[:, :, 1]
    less = x < y
    key = jnp.minimum(x, y)
    pay = jnp.where(less, px, py)
    lio = lax.broadcasted_iota(jnp.int32, (1, l2, 1), 1)
    key = lax.bitcast_convert_type(
        lax.bitcast_convert_type(key, jnp.int32) ^ ((lio & 1) << 31),
        jnp.float32)
    for j in (32, 16, 8, 4, 2, 1):
        g = _NL // (2 * j)
        ksj = key.reshape(g, 2, j, l2, r)
        psj = pay.reshape(g, 2, j, l2, r)
        a, b2 = ksj[:, 0], ksj[:, 1]
        pa_, pb_ = psj[:, 0], psj[:, 1]
        lessj = a < b2
        lo = jnp.minimum(a, b2)
        hi = jnp.maximum(a, b2)
        plo = jnp.where(lessj, pa_, pb_)
        phi = jnp.where(lessj, pb_, pa_)
        key = jnp.stack([lo, hi], axis=1).reshape(_NL, l2, r)
        pay = jnp.stack([plo, phi], axis=1).reshape(_NL, l2, r)
    return lax.bitcast_convert_type(key, jnp.int32), pay


def _topk_body(pts_row_ref, pts_all_ref, idx_ref):
    n = pts_all_ref.shape[1]
    nl2 = n // _NL
    pr = pts_row_ref[0]
    pa = pts_all_ref[0]
    inner = lax.dot_general(pa.astype(jnp.bfloat16),
                            pr.astype(jnp.bfloat16),
                            (((1,), (0,)), ((), ())),
                            preferred_element_type=jnp.float32)
    sq_r = (pr[0:1, :] * pr[0:1, :] + pr[1:2, :] * pr[1:2, :]
            + pr[2:3, :] * pr[2:3, :])
    sq_c = (pa[:, 0:1] * pa[:, 0:1] + pa[:, 1:2] * pa[:, 1:2]
            + pa[:, 2:3] * pa[:, 2:3])
    dist = (sq_r - 2.0 * inner) + sq_c

    kb = lax.bitcast_convert_type(dist.reshape(_NL, nl2, _RV), jnp.int32)
    pay = (lax.broadcasted_iota(jnp.int32, (_NL, nl2, _RV), 0) * nl2
           + lax.broadcasted_iota(jnp.int32, (_NL, nl2, _RV), 1))
    lio = lax.broadcasted_iota(jnp.int32, (1, nl2, 1), 1)
    kb = kb ^ ((lio & 1) << 31)

    kb, pay = _sort64_signed(kb, pay)
    l = nl2
    while l > 1:
        kb, pay = _merge_round_signed(kb, pay)
        l //= 2

    idx_ref[0] = jnp.transpose(pay.reshape(_NL, _RV), (1, 0))


def _topk64(points):
    b, _, n = points.shape
    nrb = n // _RV
    pts_t = jnp.transpose(points, (0, 2, 1))
    return pl.pallas_call(
        _topk_body,
        grid=(b, nrb),
        in_specs=[
            pl.BlockSpec((1, 3, _RV), lambda i, j: (i, 0, j)),
            pl.BlockSpec((1, n, 3), lambda i, j: (i, 0, 0)),
        ],
        out_specs=pl.BlockSpec((1, _RV, K64), lambda i, j: (i, j, 0)),
        out_shape=jax.ShapeDtypeStruct((b, n, K64), jnp.int32),
    )(points, pts_t)


def _node_mlp_body(f_ref, w0_ref, b0_ref, w1_ref, b1_ref, w2_ref, b2_ref,
                   m_ref):
    f = f_ref[0]
    a = lax.dot_general(f, w0_ref[...], (((0,), (1,)), ((), ())),
                        preferred_element_type=jnp.float32)
    a = jnp.maximum(a + b0_ref[...], 0.0)
    a = lax.dot_general(a, w1_ref[...], (((1,), (1,)), ((), ())),
                        preferred_element_type=jnp.float32)
    a = jnp.maximum(a + b1_ref[...], 0.0)
    a = lax.dot_general(a, w2_ref[...], (((1,), (1,)), ((), ())),
                        preferred_element_type=jnp.float32)
    m_ref[0] = a + b2_ref[...]


def _node_mlp(features, w0, b0, w1, b1, w2, b2):
    b, c, n = features.shape
    co = w2.shape[0]
    full = lambda a: pl.BlockSpec(a.shape, lambda i: (0,) * a.ndim)
    args = (w0, b0.reshape(1, -1), w1, b1.reshape(1, -1), w2,
            b2.reshape(1, -1))
    return pl.pallas_call(
        _node_mlp_body,
        grid=(b,),
        in_specs=[pl.BlockSpec((1, c, n), lambda i: (i, 0, 0))] +
                 [full(a) for a in args],
        out_specs=pl.BlockSpec((1, n, co), lambda i: (i, 0, 0)),
        out_shape=jax.ShapeDtypeStruct((b, n, co), jnp.float32),
    )(features, *args)


_NC = 2
_NS = 16
_NW = _NC * _NS
_CHUNK = 4


def _gather_max_sc(idx1, idx2, t1, t2):
    bn, c = t1.shape
    pw = bn // _NW
    nchunks = pw // _CHUNK
    rows_per_chunk = _CHUNK * KNN
    idx_rows_pw = pw * KNN // 128

    mesh = plsc.VectorSubcoreMesh(core_axis_name="c", subcore_axis_name="s")

    @functools.partial(
        pl.kernel,
        mesh=mesh,
        out_type=[jax.ShapeDtypeStruct((bn, c), jnp.float32),
                  jax.ShapeDtypeStruct((bn, c), jnp.float32)],
        scratch_types=[
            pltpu.VMEM((idx_rows_pw, 128), jnp.int32),
            pltpu.VMEM((rows_per_chunk, c), jnp.float32),
            pltpu.VMEM((rows_per_chunk, c), jnp.float32),
            pltpu.VMEM((pw, c), jnp.float32),
            pltpu.SemaphoreType.DMA,
            pltpu.SemaphoreType.DMA,
        ],
    )
    def kern(idx1_hbm, idx2_hbm, t1_hbm, t2_hbm, l1_hbm, l2_hbm,
             idx_v, rows_a, rows_b, out_v, sem_a, sem_b):
        w = lax.axis_index("s") * _NC + lax.axis_index("c")

        def compute(rows_v, cbase):
            for nloc in range(_CHUNK):
                node = cbase * _CHUNK + nloc
                for j in range(c // 16):
                    acc = rows_v[nloc * KNN, pl.ds(j * 16, 16)]
                    for k in range(1, KNN):
                        acc = jnp.maximum(
                            acc, rows_v[nloc * KNN + k, pl.ds(j * 16, 16)])
                    out_v[node, pl.ds(j * 16, 16)] = acc

        for idx_hbm, t_hbm, l_hbm in ((idx1_hbm, t1_hbm, l1_hbm),
                                      (idx2_hbm, t2_hbm, l2_hbm)):
            pltpu.sync_copy(idx_hbm.at[pl.ds(w * idx_rows_pw, idx_rows_pw)],
                            idx_v)
            pltpu.async_copy(t_hbm.at[idx_v.at[0]], rows_a, sem_a)

            def pair_body(p, _, t_hbm=t_hbm):
                c0 = p * 2
                pltpu.async_copy(t_hbm.at[idx_v.at[c0 + 1]], rows_b, sem_b)
                pltpu.make_async_copy(t_hbm.at[idx_v.at[c0]],
                                      rows_a, sem_a).wait()
                compute(rows_a, c0)

                @pl.when(p < nchunks // 2 - 1)
                def _():
                    pltpu.async_copy(t_hbm.at[idx_v.at[c0 + 2]], rows_a, sem_a)

                pltpu.make_async_copy(t_hbm.at[idx_v.at[c0 + 1]],
                                      rows_b, sem_b).wait()
                compute(rows_b, c0 + 1)
                return 0

            lax.fori_loop(0, nchunks // 2, pair_body, 0)
            pltpu.sync_copy(out_v, l_hbm.at[pl.ds(w * pw, pw)])

    return kern(idx1, idx2, t1, t2)


_NB = 1024


def _final_mlp_body(l1_ref, l2_ref, w0a_ref, w0b_ref, b0_ref, w1_ref, b1_ref,
                    w2_ref, b2_ref, out_ref):
    z = (lax.dot_general(l1_ref[0], w0a_ref[...], (((1,), (1,)), ((), ())),
                         preferred_element_type=jnp.float32) +
         lax.dot_general(l2_ref[0], w0b_ref[...], (((1,), (1,)), ((), ())),
                         preferred_element_type=jnp.float32))
    z = jnp.maximum(z + b0_ref[...], 0.0)
    z = lax.dot_general(z, w1_ref[...], (((1,), (1,)), ((), ())),
                        preferred_element_type=jnp.float32)
    z = jnp.maximum(z + b1_ref[...], 0.0)
    out = lax.dot_general(w2_ref[...], z, (((1,), (1,)), ((), ())),
                          preferred_element_type=jnp.float32)
    out_ref[0] = out + b2_ref[...]


def _final_mlp(l1, l2, w0, b0, w1, b1, w2, b2):
    b, n, c = l1.shape
    c3 = w2.shape[0]
    w0a = w0[:, :c]
    w0b = w0[:, c:]
    full = lambda a: pl.BlockSpec(a.shape, lambda i, j: (0,) * a.ndim)
    args = (w0a, w0b, b0.reshape(1, -1), w1, b1.reshape(1, -1), w2,
            b2.reshape(-1, 1))
    return pl.pallas_call(
        _final_mlp_body,
        grid=(b, n // _NB),
        in_specs=[pl.BlockSpec((1, _NB, c), lambda i, j: (i, j, 0)),
                  pl.BlockSpec((1, _NB, c), lambda i, j: (i, j, 0))] +
                 [full(a) for a in args],
        out_specs=pl.BlockSpec((1, c3, _NB), lambda i, j: (i, 0, j)),
        out_shape=jax.ShapeDtypeStruct((b, c3, n), jnp.float32),
    )(l1, l2, *args)


def kernel(points, features, m1_w0, m1_b0, m1_w1, m1_b1, m1_w2, m1_b2,
           m2_w0, m2_b0, m2_w1, m2_b1, m2_w2, m2_b2,
           mm_w0, mm_b0, mm_w1, mm_b1, mm_w2, mm_b2):
    b, c, n = features.shape
    bn = b * n

    idx64 = _topk64(points)

    m1 = _node_mlp(features, m1_w0, m1_b0, m1_w1, m1_b1, m1_w2, m1_b2)
    m2 = _node_mlp(features, m2_w0, m2_b0, m2_w1, m2_b1, m2_w2, m2_b2)

    offs = (jnp.arange(b, dtype=jnp.int32) * n)[:, None, None]
    idx1 = (idx64[:, :, :KNN] + offs).reshape(bn * KNN // 128, 128)
    idx2 = (idx64[:, :, ::DIL] + offs).reshape(bn * KNN // 128, 128)

    l1, l2 = _gather_max_sc(idx1, idx2,
                            m1.reshape(bn, c), m2.reshape(bn, c))

    return _final_mlp(l1.reshape(b, n, c), l2.reshape(b, n, c),
                      mm_w0, mm_b0, mm_w1, mm_b1, mm_w2, mm_b2)


# speedup vs baseline: 2.5430x; 1.5818x over previous
import functools

import jax
import jax.numpy as jnp
from jax import lax
from jax.experimental import pallas as pl
from jax.experimental.pallas import tpu as pltpu
from jax.experimental.pallas import tpu_sc as plsc

KNN = 32
DIL = 2
K64 = KNN * DIL


_RV = 128
_NL = 64


def _bf16_round(x):
    r = lax.bitcast_convert_type(x, jnp.int32)
    r = (r + 0x7FFF + ((r >> 16) & 1)) & ~0xFFFF
    return lax.bitcast_convert_type(r, jnp.float32)


def _neg_bits(kb):
    return kb ^ (-2147483648)


def _sort64_signed(kb, pay):
    l, r = kb.shape[1], kb.shape[2]
    io = lax.broadcasted_iota(jnp.int32, (_NL, 1, 1), 0)
    prev = None
    for k2 in (2, 4, 8, 16, 32, 64):
        pat = (io & k2) << (31 - k2.bit_length() + 1)
        kb = kb ^ pat if prev is None else kb ^ (prev ^ pat)
        prev = pat
        key = lax.bitcast_convert_type(kb, jnp.float32)
        j = k2 // 2
        while j >= 1:
            g = _NL // (2 * j)
            ks = key.reshape(g, 2, j, l, r)
            ps = pay.reshape(g, 2, j, l, r)
            a, b2 = ks[:, 0], ks[:, 1]
            pa_, pb_ = ps[:, 0], ps[:, 1]
            less = a < b2
            lo = jnp.minimum(a, b2)
            hi = jnp.maximum(a, b2)
            plo = jnp.where(less, pa_, pb_)
            phi = jnp.where(less, pb_, pa_)
            key = jnp.stack([lo, hi], axis=1).reshape(_NL, l, r)
            pay = jnp.stack([plo, phi], axis=1).reshape(_NL, l, r)
            j //= 2
        kb = lax.bitcast_convert_type(key, jnp.int32)
    return kb, pay


def _merge_round_signed(kb, pay):
    l, r = kb.shape[1], kb.shape[2]
    l2 = l // 2
    ks = kb.reshape(_NL, l2, 2, r)
    ps = pay.reshape(_NL, l2, 2, r)
    x = lax.bitcast_convert_type(ks[:, :, 0], jnp.float32)
    y = lax.bitcast_convert_type(_neg_bits(ks[:, :, 1]), jnp.float32)
    px, py = ps[:, :, 0], ps[:, :, 1]
    less = x < y
    key = jnp.minimum(x, y)
    pay = jnp.where(less, px, py)
    lio = lax.broadcasted_iota(jnp.int32, (1, l2, 1), 1)
    key = lax.bitcast_convert_type(
        lax.bitcast_convert_type(key, jnp.int32) ^ ((lio & 1) << 31),
        jnp.float32)
    for j in (32, 16, 8, 4, 2, 1):
        g = _NL // (2 * j)
        ksj = key.reshape(g, 2, j, l2, r)
        psj = pay.reshape(g, 2, j, l2, r)
        a, b2 = ksj[:, 0], ksj[:, 1]
        pa_, pb_ = psj[:, 0], psj[:, 1]
        lessj = a < b2
        lo = jnp.minimum(a, b2)
        hi = jnp.maximum(a, b2)
        plo = jnp.where(lessj, pa_, pb_)
        phi = jnp.where(lessj, pb_, pa_)
        key = jnp.stack([lo, hi], axis=1).reshape(_NL, l2, r)
        pay = jnp.stack([plo, phi], axis=1).reshape(_NL, l2, r)
    return lax.bitcast_convert_type(key, jnp.int32), pay


def _topk_body(pts_row_ref, pts_all_ref, idx_ref):
    n = pts_all_ref.shape[1]
    nl2 = n // _NL
    pr = pts_row_ref[0]
    pa = pts_all_ref[0]
    inner = lax.dot_general(pa.astype(jnp.bfloat16),
                            pr.astype(jnp.bfloat16),
                            (((1,), (0,)), ((), ())),
                            preferred_element_type=jnp.float32)
    sq_r = (pr[0:1, :] * pr[0:1, :] + pr[1:2, :] * pr[1:2, :]
            + pr[2:3, :] * pr[2:3, :])
    sq_c = (pa[:, 0:1] * pa[:, 0:1] + pa[:, 1:2] * pa[:, 1:2]
            + pa[:, 2:3] * pa[:, 2:3])
    dist = (sq_r - 2.0 * inner) + sq_c

    kb = lax.bitcast_convert_type(dist.reshape(_NL, nl2, _RV), jnp.int32)
    pay = (lax.broadcasted_iota(jnp.int32, (_NL, nl2, _RV), 0) * nl2
           + lax.broadcasted_iota(jnp.int32, (_NL, nl2, _RV), 1))
    lio = lax.broadcasted_iota(jnp.int32, (1, nl2, 1), 1)
    kb = kb ^ ((lio & 1) << 31)

    kb, pay = _sort64_signed(kb, pay)
    l = nl2
    while l > 1:
        kb, pay = _merge_round_signed(kb, pay)
        l //= 2

    idx_ref[0] = jnp.transpose(pay.reshape(_NL, _RV), (1, 0))


def _topk64(points):
    b, _, n = points.shape
    nrb = n // _RV
    pts_t = jnp.transpose(points, (0, 2, 1))
    return pl.pallas_call(
        _topk_body,
        grid=(b, nrb),
        in_specs=[
            pl.BlockSpec((1, 3, _RV), lambda i, j: (i, 0, j)),
            pl.BlockSpec((1, n, 3), lambda i, j: (i, 0, 0)),
        ],
        out_specs=pl.BlockSpec((1, _RV, K64), lambda i, j: (i, j, 0)),
        out_shape=jax.ShapeDtypeStruct((b, n, K64), jnp.int32),
    )(points, pts_t)


def _node_mlp_body(f_ref, w0_ref, b0_ref, w1_ref, b1_ref, w2_ref, b2_ref,
                   m_ref):
    f = f_ref[0]
    a = lax.dot_general(f, w0_ref[...], (((0,), (1,)), ((), ())),
                        preferred_element_type=jnp.float32)
    a = jnp.maximum(a + b0_ref[...], 0.0)
    a = lax.dot_general(a, w1_ref[...], (((1,), (1,)), ((), ())),
                        preferred_element_type=jnp.float32)
    a = jnp.maximum(a + b1_ref[...], 0.0)
    a = lax.dot_general(a, w2_ref[...], (((1,), (1,)), ((), ())),
                        preferred_element_type=jnp.float32)
    m_ref[0] = a + b2_ref[...]


def _node_mlp(features, w0, b0, w1, b1, w2, b2):
    b, c, n = features.shape
    co = w2.shape[0]
    full = lambda a: pl.BlockSpec(a.shape, lambda i: (0,) * a.ndim)
    args = (w0, b0.reshape(1, -1), w1, b1.reshape(1, -1), w2,
            b2.reshape(1, -1))
    return pl.pallas_call(
        _node_mlp_body,
        grid=(b,),
        in_specs=[pl.BlockSpec((1, c, n), lambda i: (i, 0, 0))] +
                 [full(a) for a in args],
        out_specs=pl.BlockSpec((1, n, co), lambda i: (i, 0, 0)),
        out_shape=jax.ShapeDtypeStruct((b, n, co), jnp.float32),
    )(features, *args)


_NC = 2
_NS = 16
_NW = _NC * _NS
_CHUNK = 4


def _gather_max_sc(idx1, idx2, t1, t2):
    bn, c = t1.shape
    pw = bn // _NW
    nchunks = pw // _CHUNK
    rows_per_chunk = _CHUNK * KNN
    idx_rows_pw = pw * KNN // 128

    mesh = plsc.VectorSubcoreMesh(core_axis_name="c", subcore_axis_name="s")

    @functools.partial(
        pl.kernel,
        mesh=mesh,
        out_type=[jax.ShapeDtypeStruct((bn, c), jnp.float32),
                  jax.ShapeDtypeStruct((bn, c), jnp.float32)],
        scratch_types=[
            pltpu.VMEM((idx_rows_pw, 128), jnp.int32),
            pltpu.VMEM((rows_per_chunk, c), jnp.float32),
            pltpu.VMEM((rows_per_chunk, c), jnp.float32),
            pltpu.VMEM((pw, c), jnp.float32),
            pltpu.SemaphoreType.DMA,
            pltpu.SemaphoreType.DMA,
        ],
    )
    def kern(idx1_hbm, idx2_hbm, t1_hbm, t2_hbm, l1_hbm, l2_hbm,
             idx_v, rows_a, rows_b, out_v, sem_a, sem_b):
        w = lax.axis_index("s") * _NC + lax.axis_index("c")

        def compute(rows_v, cbase):
            for nloc in range(_CHUNK):
                node = cbase * _CHUNK + nloc
                for j in range(c // 16):
                    acc = rows_v[nloc * KNN, pl.ds(j * 16, 16)]
                    for k in range(1, KNN):
                        acc = jnp.maximum(
                            acc, rows_v[nloc * KNN + k, pl.ds(j * 16, 16)])
                    out_v[node, pl.ds(j * 16, 16)] = acc

        for idx_hbm, t_hbm, l_hbm in ((idx1_hbm, t1_hbm, l1_hbm),
                                      (idx2_hbm, t2_hbm, l2_hbm)):
            pltpu.sync_copy(idx_hbm.at[pl.ds(w * idx_rows_pw, idx_rows_pw)],
                            idx_v)
            pltpu.async_copy(t_hbm.at[idx_v.at[0]], rows_a, sem_a)

            def pair_body(p, _, t_hbm=t_hbm):
                c0 = p * 2
                pltpu.async_copy(t_hbm.at[idx_v.at[c0 + 1]], rows_b, sem_b)
                pltpu.make_async_copy(t_hbm.at[idx_v.at[c0]],
                                      rows_a, sem_a).wait()
                compute(rows_a, c0)

                @pl.when(p < nchunks // 2 - 1)
                def _():
                    pltpu.async_copy(t_hbm.at[idx_v.at[c0 + 2]], rows_a, sem_a)

                pltpu.make_async_copy(t_hbm.at[idx_v.at[c0 + 1]],
                                      rows_b, sem_b).wait()
                compute(rows_b, c0 + 1)
                return 0

            lax.fori_loop(0, nchunks // 2, pair_body, 0)
            pltpu.sync_copy(out_v, l_hbm.at[pl.ds(w * pw, pw)])

    return kern(idx1, idx2, t1, t2)


_NB = 1024


def _final_mlp_body(l1_ref, l2_ref, w0a_ref, w0b_ref, b0_ref, w1_ref, b1_ref,
                    w2_ref, b2_ref, out_ref):
    z = (lax.dot_general(l1_ref[0], w0a_ref[...], (((1,), (1,)), ((), ())),
                         preferred_element_type=jnp.float32) +
         lax.dot_general(l2_ref[0], w0b_ref[...], (((1,), (1,)), ((), ())),
                         preferred_element_type=jnp.float32))
    z = jnp.maximum(z + b0_ref[...], 0.0)
    z = lax.dot_general(z, w1_ref[...], (((1,), (1,)), ((), ())),
                        preferred_element_type=jnp.float32)
    z = jnp.maximum(z + b1_ref[...], 0.0)
    out = lax.dot_general(w2_ref[...], z, (((1,), (1,)), ((), ())),
                          preferred_element_type=jnp.float32)
    out_ref[0] = out + b2_ref[...]


def _final_mlp(l1, l2, w0, b0, w1, b1, w2, b2):
    b, n, c = l1.shape
    c3 = w2.shape[0]
    w0a = w0[:, :c]
    w0b = w0[:, c:]
    full = lambda a: pl.BlockSpec(a.shape, lambda i, j: (0,) * a.ndim)
    args = (w0a, w0b, b0.reshape(1, -1), w1, b1.reshape(1, -1), w2,
            b2.reshape(-1, 1))
    return pl.pallas_call(
        _final_mlp_body,
        grid=(b, n // _NB),
        in_specs=[pl.BlockSpec((1, _NB, c), lambda i, j: (i, j, 0)),
                  pl.BlockSpec((1, _NB, c), lambda i, j: (i, j, 0))] +
                 [full(a) for a in args],
        out_specs=pl.BlockSpec((1, c3, _NB), lambda i, j: (i, 0, j)),
        out_shape=jax.ShapeDtypeStruct((b, c3, n), jnp.float32),
    )(l1, l2, *args)


def kernel(points, features, m1_w0, m1_b0, m1_w1, m1_b1, m1_w2, m1_b2,
           m2_w0, m2_b0, m2_w1, m2_b1, m2_w2, m2_b2,
           mm_w0, mm_b0, mm_w1, mm_b1, mm_w2, mm_b2):
    b, c, n = features.shape

    outs = []
    for bi in range(b):
        pts = lax.slice_in_dim(points, bi, bi + 1, axis=0)
        fts = lax.slice_in_dim(features, bi, bi + 1, axis=0)
        idx64 = _topk64(pts)
        m1 = _node_mlp(fts, m1_w0, m1_b0, m1_w1, m1_b1, m1_w2, m1_b2)
        m2 = _node_mlp(fts, m2_w0, m2_b0, m2_w1, m2_b1, m2_w2, m2_b2)
        idx1 = idx64[:, :, :KNN].reshape(n * KNN // 128, 128)
        idx2 = idx64[:, :, ::DIL].reshape(n * KNN // 128, 128)
        l1, l2 = _gather_max_sc(idx1, idx2,
                                m1.reshape(n, c), m2.reshape(n, c))
        outs.append(_final_mlp(l1.reshape(1, n, c), l2.reshape(1, n, c),
                               mm_w0, mm_b0, mm_w1, mm_b1, mm_w2, mm_b2))
    return jnp.concatenate(outs, axis=0)


# speedup vs baseline: 2.5434x; 1.0002x over previous
import functools

import jax
import jax.numpy as jnp
from jax import lax
from jax.experimental import pallas as pl
from jax.experimental.pallas import tpu as pltpu
from jax.experimental.pallas import tpu_sc as plsc

KNN = 32
DIL = 2
K64 = KNN * DIL


_RV = 128
_NL = 64


def _bf16_round(x):
    r = lax.bitcast_convert_type(x, jnp.int32)
    r = (r + 0x7FFF + ((r >> 16) & 1)) & ~0xFFFF
    return lax.bitcast_convert_type(r, jnp.float32)


def _neg_bits(kb):
    return kb ^ (-2147483648)


def _cx_tree(key, pay):
    m = key.shape[1]
    if m == 1:
        return [(key, pay)]
    j = m // 2
    a, b2 = key[:, :j], key[:, j:]
    pa_, pb_ = pay[:, :j], pay[:, j:]
    less = a < b2
    lo = jnp.minimum(a, b2)
    hi = jnp.maximum(a, b2)
    plo = jnp.where(less, pa_, pb_)
    phi = jnp.where(less, pb_, pa_)
    return _cx_tree(lo, plo) + _cx_tree(hi, phi)


def _bitonic_merge_blocks(key, pay, k2):
    l, r = key.shape[1], key.shape[2]
    nb = _NL // k2
    pieces = _cx_tree(key.reshape(nb, k2, l, r), pay.reshape(nb, k2, l, r))
    key = jnp.concatenate([k for k, _ in pieces], axis=1).reshape(_NL, l, r)
    pay = jnp.concatenate([p for _, p in pieces], axis=1).reshape(_NL, l, r)
    return key, pay


def _sort64_signed(kb, pay):
    io = lax.broadcasted_iota(jnp.int32, (_NL, 1, 1), 0)
    prev = None
    for k2 in (2, 4, 8, 16, 32, 64):
        pat = (io & k2) << (31 - k2.bit_length() + 1)
        kb = kb ^ pat if prev is None else kb ^ (prev ^ pat)
        prev = pat
        key = lax.bitcast_convert_type(kb, jnp.float32)
        key, pay = _bitonic_merge_blocks(key, pay, k2)
        kb = lax.bitcast_convert_type(key, jnp.int32)
    return kb, pay


def _merge_round_signed(kb, pay):
    l, r = kb.shape[1], kb.shape[2]
    l2 = l // 2
    ks = kb.reshape(_NL, l2, 2, r)
    ps = pay.reshape(_NL, l2, 2, r)
    x = lax.bitcast_convert_type(ks[:, :, 0], jnp.float32)
    y = lax.bitcast_convert_type(_neg_bits(ks[:, :, 1]), jnp.float32)
    px, py = ps[:, :, 0], ps[:, :, 1]
    less = x < y
    key = jnp.minimum(x, y)
    pay = jnp.where(less, px, py)
    lio = lax.broadcasted_iota(jnp.int32, (1, l2, 1), 1)
    key = lax.bitcast_convert_type(
        lax.bitcast_convert_type(key, jnp.int32) ^ ((lio & 1) << 31),
        jnp.float32)
    key, pay = _bitonic_merge_blocks(key, pay, _NL)
    return lax.bitcast_convert_type(key, jnp.int32), pay


def _topk_body(pts_row_ref, pts_all_ref, idx_ref):
    n = pts_all_ref.shape[1]
    nl2 = n // _NL
    pr = pts_row_ref[0]
    pa = pts_all_ref[0]
    inner = lax.dot_general(pa.astype(jnp.bfloat16),
                            pr.astype(jnp.bfloat16),
                            (((1,), (0,)), ((), ())),
                            preferred_element_type=jnp.float32)
    sq_r = (pr[0:1, :] * pr[0:1, :] + pr[1:2, :] * pr[1:2, :]
            + pr[2:3, :] * pr[2:3, :])
    sq_c = (pa[:, 0:1] * pa[:, 0:1] + pa[:, 1:2] * pa[:, 1:2]
            + pa[:, 2:3] * pa[:, 2:3])
    dist = (sq_r - 2.0 * inner) + sq_c

    kb = lax.bitcast_convert_type(dist.reshape(_NL, nl2, _RV), jnp.int32)
    pay = (lax.broadcasted_iota(jnp.int32, (_NL, nl2, _RV), 0) * nl2
           + lax.broadcasted_iota(jnp.int32, (_NL, nl2, _RV), 1))
    lio = lax.broadcasted_iota(jnp.int32, (1, nl2, 1), 1)
    kb = kb ^ ((lio & 1) << 31)

    kb, pay = _sort64_signed(kb, pay)
    l = nl2
    while l > 1:
        kb, pay = _merge_round_signed(kb, pay)
        l //= 2

    idx_ref[0] = jnp.transpose(pay.reshape(_NL, _RV), (1, 0))


def _topk64(points):
    b, _, n = points.shape
    nrb = n // _RV
    pts_t = jnp.transpose(points, (0, 2, 1))
    return pl.pallas_call(
        _topk_body,
        grid=(b, nrb),
        in_specs=[
            pl.BlockSpec((1, 3, _RV), lambda i, j: (i, 0, j)),
            pl.BlockSpec((1, n, 3), lambda i, j: (i, 0, 0)),
        ],
        out_specs=pl.BlockSpec((1, _RV, K64), lambda i, j: (i, j, 0)),
        out_shape=jax.ShapeDtypeStruct((b, n, K64), jnp.int32),
    )(points, pts_t)


def _node_mlp_body(f_ref, w0_ref, b0_ref, w1_ref, b1_ref, w2_ref, b2_ref,
                   m_ref):
    f = f_ref[0]
    a = lax.dot_general(f, w0_ref[...], (((0,), (1,)), ((), ())),
                        preferred_element_type=jnp.float32)
    a = jnp.maximum(a + b0_ref[...], 0.0)
    a = lax.dot_general(a, w1_ref[...], (((1,), (1,)), ((), ())),
                        preferred_element_type=jnp.float32)
    a = jnp.maximum(a + b1_ref[...], 0.0)
    a = lax.dot_general(a, w2_ref[...], (((1,), (1,)), ((), ())),
                        preferred_element_type=jnp.float32)
    m_ref[0] = a + b2_ref[...]


def _node_mlp(features, w0, b0, w1, b1, w2, b2):
    b, c, n = features.shape
    co = w2.shape[0]
    full = lambda a: pl.BlockSpec(a.shape, lambda i: (0,) * a.ndim)
    args = (w0, b0.reshape(1, -1), w1, b1.reshape(1, -1), w2,
            b2.reshape(1, -1))
    return pl.pallas_call(
        _node_mlp_body,
        grid=(b,),
        in_specs=[pl.BlockSpec((1, c, n), lambda i: (i, 0, 0))] +
                 [full(a) for a in args],
        out_specs=pl.BlockSpec((1, n, co), lambda i: (i, 0, 0)),
        out_shape=jax.ShapeDtypeStruct((b, n, co), jnp.float32),
    )(features, *args)


_NC = 2
_NS = 16
_NW = _NC * _NS
_CHUNK = 4


def _gather_max_sc(idx1, idx2, t1, t2):
    bn, c = t1.shape
    pw = bn // _NW
    nchunks = pw // _CHUNK
    rows_per_chunk = _CHUNK * KNN
    idx_rows_pw = pw * KNN // 128

    mesh = plsc.VectorSubcoreMesh(core_axis_name="c", subcore_axis_name="s")

    @functools.partial(
        pl.kernel,
        mesh=mesh,
        out_type=[jax.ShapeDtypeStruct((bn, c), jnp.float32),
                  jax.ShapeDtypeStruct((bn, c), jnp.float32)],
        scratch_types=[
            pltpu.VMEM((idx_rows_pw, 128), jnp.int32),
            pltpu.VMEM((rows_per_chunk, c), jnp.float32),
            pltpu.VMEM((rows_per_chunk, c), jnp.float32),
            pltpu.VMEM((pw, c), jnp.float32),
            pltpu.SemaphoreType.DMA,
            pltpu.SemaphoreType.DMA,
        ],
    )
    def kern(idx1_hbm, idx2_hbm, t1_hbm, t2_hbm, l1_hbm, l2_hbm,
             idx_v, rows_a, rows_b, out_v, sem_a, sem_b):
        w = lax.axis_index("s") * _NC + lax.axis_index("c")

        def compute(rows_v, cbase):
            for nloc in range(_CHUNK):
                node = cbase * _CHUNK + nloc
                for j in range(c // 16):
                    acc = rows_v[nloc * KNN, pl.ds(j * 16, 16)]
                    for k in range(1, KNN):
                        acc = jnp.maximum(
                            acc, rows_v[nloc * KNN + k, pl.ds(j * 16, 16)])
                    out_v[node, pl.ds(j * 16, 16)] = acc

        for idx_hbm, t_hbm, l_hbm in ((idx1_hbm, t1_hbm, l1_hbm),
                                      (idx2_hbm, t2_hbm, l2_hbm)):
            pltpu.sync_copy(idx_hbm.at[pl.ds(w * idx_rows_pw, idx_rows_pw)],
                            idx_v)
            pltpu.async_copy(t_hbm.at[idx_v.at[0]], rows_a, sem_a)

            def pair_body(p, _, t_hbm=t_hbm):
                c0 = p * 2
                pltpu.async_copy(t_hbm.at[idx_v.at[c0 + 1]], rows_b, sem_b)
                pltpu.make_async_copy(t_hbm.at[idx_v.at[c0]],
                                      rows_a, sem_a).wait()
                compute(rows_a, c0)

                @pl.when(p < nchunks // 2 - 1)
                def _():
                    pltpu.async_copy(t_hbm.at[idx_v.at[c0 + 2]], rows_a, sem_a)

                pltpu.make_async_copy(t_hbm.at[idx_v.at[c0 + 1]],
                                      rows_b, sem_b).wait()
                compute(rows_b, c0 + 1)
                return 0

            lax.fori_loop(0, nchunks // 2, pair_body, 0)
            pltpu.sync_copy(out_v, l_hbm.at[pl.ds(w * pw, pw)])

    return kern(idx1, idx2, t1, t2)


_NB = 1024


def _final_mlp_body(l1_ref, l2_ref, w0a_ref, w0b_ref, b0_ref, w1_ref, b1_ref,
                    w2_ref, b2_ref, out_ref):
    z = (lax.dot_general(l1_ref[0], w0a_ref[...], (((1,), (1,)), ((), ())),
                         preferred_element_type=jnp.float32) +
         lax.dot_general(l2_ref[0], w0b_ref[...], (((1,), (1,)), ((), ())),
                         preferred_element_type=jnp.float32))
    z = jnp.maximum(z + b0_ref[...], 0.0)
    z = lax.dot_general(z, w1_ref[...], (((1,), (1,)), ((), ())),
                        preferred_element_type=jnp.float32)
    z = jnp.maximum(z + b1_ref[...], 0.0)
    out = lax.dot_general(w2_ref[...], z, (((1,), (1,)), ((), ())),
                          preferred_element_type=jnp.float32)
    out_ref[0] = out + b2_ref[...]


def _final_mlp(l1, l2, w0, b0, w1, b1, w2, b2):
    b, n, c = l1.shape
    c3 = w2.shape[0]
    w0a = w0[:, :c]
    w0b = w0[:, c:]
    full = lambda a: pl.BlockSpec(a.shape, lambda i, j: (0,) * a.ndim)
    args = (w0a, w0b, b0.reshape(1, -1), w1, b1.reshape(1, -1), w2,
            b2.reshape(-1, 1))
    return pl.pallas_call(
        _final_mlp_body,
        grid=(b, n // _NB),
        in_specs=[pl.BlockSpec((1, _NB, c), lambda i, j: (i, j, 0)),
                  pl.BlockSpec((1, _NB, c), lambda i, j: (i, j, 0))] +
                 [full(a) for a in args],
        out_specs=pl.BlockSpec((1, c3, _NB), lambda i, j: (i, 0, j)),
        out_shape=jax.ShapeDtypeStruct((b, c3, n), jnp.float32),
    )(l1, l2, *args)


def kernel(points, features, m1_w0, m1_b0, m1_w1, m1_b1, m1_w2, m1_b2,
           m2_w0, m2_b0, m2_w1, m2_b1, m2_w2, m2_b2,
           mm_w0, mm_b0, mm_w1, mm_b1, mm_w2, mm_b2):
    b, c, n = features.shape

    outs = []
    for bi in range(b):
        pts = lax.slice_in_dim(points, bi, bi + 1, axis=0)
        fts = lax.slice_in_dim(features, bi, bi + 1, axis=0)
        idx64 = _topk64(pts)
        m1 = _node_mlp(fts, m1_w0, m1_b0, m1_w1, m1_b1, m1_w2, m1_b2)
        m2 = _node_mlp(fts, m2_w0, m2_b0, m2_w1, m2_b1, m2_w2, m2_b2)
        idx1 = idx64[:, :, :KNN].reshape(n * KNN // 128, 128)
        idx2 = idx64[:, :, ::DIL].reshape(n * KNN // 128, 128)
        l1, l2 = _gather_max_sc(idx1, idx2,
                                m1.reshape(n, c), m2.reshape(n, c))
        outs.append(_final_mlp(l1.reshape(1, n, c), l2.reshape(1, n, c),
                               mm_w0, mm_b0, mm_w1, mm_b1, mm_w2, mm_b2))
    return jnp.concatenate(outs, axis=0)


# speedup vs baseline: 6.8126x; 2.6786x over previous
import functools

import jax
import jax.numpy as jnp
from jax import lax
from jax.experimental import pallas as pl
from jax.experimental.pallas import tpu as pltpu
from jax.experimental.pallas import tpu_sc as plsc

KNN = 32
DIL = 2
K64 = KNN * DIL


_RV = 128
_NL = 64


def _bf16_round(x):
    r = lax.bitcast_convert_type(x, jnp.int32)
    r = (r + 0x7FFF + ((r >> 16) & 1)) & ~0xFFFF
    return lax.bitcast_convert_type(r, jnp.float32)


def _neg_bits(kb):
    return kb ^ (-2147483648)


def _cx_tree(key, pay):
    m = key.shape[1]
    if m == 1:
        return [(key, pay)]
    j = m // 2
    a, b2 = key[:, :j], key[:, j:]
    pa_, pb_ = pay[:, :j], pay[:, j:]
    less = a < b2
    lo = jnp.minimum(a, b2)
    hi = jnp.maximum(a, b2)
    plo = jnp.where(less, pa_, pb_)
    phi = jnp.where(less, pb_, pa_)
    return _cx_tree(lo, plo) + _cx_tree(hi, phi)


def _bitonic_merge_blocks(key, pay, k2):
    l, r = key.shape[1], key.shape[2]
    nb = _NL // k2
    pieces = _cx_tree(key.reshape(nb, k2, l, r), pay.reshape(nb, k2, l, r))
    key = jnp.concatenate([k for k, _ in pieces], axis=1).reshape(_NL, l, r)
    pay = jnp.concatenate([p for _, p in pieces], axis=1).reshape(_NL, l, r)
    return key, pay


def _sort64_signed(kb, pay):
    io = lax.broadcasted_iota(jnp.int32, (_NL, 1, 1), 0)
    prev = None
    for k2 in (2, 4, 8, 16, 32, 64):
        pat = (io & k2) << (31 - k2.bit_length() + 1)
        kb = kb ^ pat if prev is None else kb ^ (prev ^ pat)
        prev = pat
        key = lax.bitcast_convert_type(kb, jnp.float32)
        key, pay = _bitonic_merge_blocks(key, pay, k2)
        kb = lax.bitcast_convert_type(key, jnp.int32)
    return kb, pay


def _merge_round_signed(kb, pay):
    l, r = kb.shape[1], kb.shape[2]
    l2 = l // 2
    x = lax.bitcast_convert_type(kb[:, :l2], jnp.float32)
    y = lax.bitcast_convert_type(_neg_bits(kb[:, l2:]), jnp.float32)
    px, py = pay[:, :l2], pay[:, l2:]
    less = x < y
    key = jnp.minimum(x, y)
    pay = jnp.where(less, px, py)
    if l2 > 1:
        q = l2 // 2
        lio = lax.broadcasted_iota(jnp.int32, (1, l2, 1), 1)
        key = lax.bitcast_convert_type(
            lax.bitcast_convert_type(key, jnp.int32)
            ^ ((lio & q) << (31 - q.bit_length() + 1)),
            jnp.float32)
    key, pay = _bitonic_merge_blocks(key, pay, _NL)
    return lax.bitcast_convert_type(key, jnp.int32), pay


def _topk_body(pts_row_ref, pts_all_ref, idx_ref):
    n = pts_all_ref.shape[1]
    nl2 = n // _NL
    pr = pts_row_ref[0]
    pa = pts_all_ref[0]
    inner = lax.dot_general(pa.astype(jnp.bfloat16),
                            pr.astype(jnp.bfloat16),
                            (((1,), (0,)), ((), ())),
                            preferred_element_type=jnp.float32)
    sq_r = (pr[0:1, :] * pr[0:1, :] + pr[1:2, :] * pr[1:2, :]
            + pr[2:3, :] * pr[2:3, :])
    sq_c = (pa[:, 0:1] * pa[:, 0:1] + pa[:, 1:2] * pa[:, 1:2]
            + pa[:, 2:3] * pa[:, 2:3])
    dist = (sq_r - 2.0 * inner) + sq_c

    kb = lax.bitcast_convert_type(dist.reshape(_NL, nl2, _RV), jnp.int32)
    pay = (lax.broadcasted_iota(jnp.int32, (_NL, nl2, _RV), 0) * nl2
           + lax.broadcasted_iota(jnp.int32, (_NL, nl2, _RV), 1))
    lio = lax.broadcasted_iota(jnp.int32, (1, nl2, 1), 1)
    h = nl2 // 2
    kb = kb ^ ((lio & h) << (31 - h.bit_length() + 1))

    kb, pay = _sort64_signed(kb, pay)
    l = nl2
    while l > 1:
        kb, pay = _merge_round_signed(kb, pay)
        l //= 2

    idx_ref[0] = jnp.transpose(pay.reshape(_NL, _RV), (1, 0))


def _topk64(points):
    b, _, n = points.shape
    nrb = n // _RV
    pts_t = jnp.transpose(points, (0, 2, 1))
    return pl.pallas_call(
        _topk_body,
        grid=(b, nrb),
        in_specs=[
            pl.BlockSpec((1, 3, _RV), lambda i, j: (i, 0, j)),
            pl.BlockSpec((1, n, 3), lambda i, j: (i, 0, 0)),
        ],
        out_specs=pl.BlockSpec((1, _RV, K64), lambda i, j: (i, j, 0)),
        out_shape=jax.ShapeDtypeStruct((b, n, K64), jnp.int32),
    )(points, pts_t)


def _node_mlp_body(f_ref, w0_ref, b0_ref, w1_ref, b1_ref, w2_ref, b2_ref,
                   m_ref):
    f = f_ref[0]
    a = lax.dot_general(f, w0_ref[...], (((0,), (1,)), ((), ())),
                        preferred_element_type=jnp.float32)
    a = jnp.maximum(a + b0_ref[...], 0.0)
    a = lax.dot_general(a, w1_ref[...], (((1,), (1,)), ((), ())),
                        preferred_element_type=jnp.float32)
    a = jnp.maximum(a + b1_ref[...], 0.0)
    a = lax.dot_general(a, w2_ref[...], (((1,), (1,)), ((), ())),
                        preferred_element_type=jnp.float32)
    m_ref[0] = a + b2_ref[...]


def _node_mlp(features, w0, b0, w1, b1, w2, b2):
    b, c, n = features.shape
    co = w2.shape[0]
    full = lambda a: pl.BlockSpec(a.shape, lambda i: (0,) * a.ndim)
    args = (w0, b0.reshape(1, -1), w1, b1.reshape(1, -1), w2,
            b2.reshape(1, -1))
    return pl.pallas_call(
        _node_mlp_body,
        grid=(b,),
        in_specs=[pl.BlockSpec((1, c, n), lambda i: (i, 0, 0))] +
                 [full(a) for a in args],
        out_specs=pl.BlockSpec((1, n, co), lambda i: (i, 0, 0)),
        out_shape=jax.ShapeDtypeStruct((b, n, co), jnp.float32),
    )(features, *args)


_NC = 2
_NS = 16
_NW = _NC * _NS
_CHUNK = 4


def _gather_max_sc(idx1, idx2, t1, t2):
    bn, c = t1.shape
    pw = bn // _NW
    nchunks = pw // _CHUNK
    rows_per_chunk = _CHUNK * KNN
    idx_rows_pw = pw * KNN // 128

    mesh = plsc.VectorSubcoreMesh(core_axis_name="c", subcore_axis_name="s")

    @functools.partial(
        pl.kernel,
        mesh=mesh,
        out_type=[jax.ShapeDtypeStruct((bn, c), jnp.float32),
                  jax.ShapeDtypeStruct((bn, c), jnp.float32)],
        scratch_types=[
            pltpu.VMEM((idx_rows_pw, 128), jnp.int32),
            pltpu.VMEM((rows_per_chunk, c), jnp.float32),
            pltpu.VMEM((rows_per_chunk, c), jnp.float32),
            pltpu.VMEM((pw, c), jnp.float32),
            pltpu.SemaphoreType.DMA,
            pltpu.SemaphoreType.DMA,
        ],
    )
    def kern(idx1_hbm, idx2_hbm, t1_hbm, t2_hbm, l1_hbm, l2_hbm,
             idx_v, rows_a, rows_b, out_v, sem_a, sem_b):
        w = lax.axis_index("s") * _NC + lax.axis_index("c")

        def compute(rows_v, cbase):
            for nloc in range(_CHUNK):
                node = cbase * _CHUNK + nloc
                for j in range(c // 16):
                    acc = rows_v[nloc * KNN, pl.ds(j * 16, 16)]
                    for k in range(1, KNN):
                        acc = jnp.maximum(
                            acc, rows_v[nloc * KNN + k, pl.ds(j * 16, 16)])
                    out_v[node, pl.ds(j * 16, 16)] = acc

        for idx_hbm, t_hbm, l_hbm in ((idx1_hbm, t1_hbm, l1_hbm),
                                      (idx2_hbm, t2_hbm, l2_hbm)):
            pltpu.sync_copy(idx_hbm.at[pl.ds(w * idx_rows_pw, idx_rows_pw)],
                            idx_v)
            pltpu.async_copy(t_hbm.at[idx_v.at[0]], rows_a, sem_a)

            def pair_body(p, _, t_hbm=t_hbm):
                c0 = p * 2
                pltpu.async_copy(t_hbm.at[idx_v.at[c0 + 1]], rows_b, sem_b)
                pltpu.make_async_copy(t_hbm.at[idx_v.at[c0]],
                                      rows_a, sem_a).wait()
                compute(rows_a, c0)

                @pl.when(p < nchunks // 2 - 1)
                def _():
                    pltpu.async_copy(t_hbm.at[idx_v.at[c0 + 2]], rows_a, sem_a)

                pltpu.make_async_copy(t_hbm.at[idx_v.at[c0 + 1]],
                                      rows_b, sem_b).wait()
                compute(rows_b, c0 + 1)
                return 0

            lax.fori_loop(0, nchunks // 2, pair_body, 0)
            pltpu.sync_copy(out_v, l_hbm.at[pl.ds(w * pw, pw)])

    return kern(idx1, idx2, t1, t2)


_NB = 1024


def _final_mlp_body(l1_ref, l2_ref, w0a_ref, w0b_ref, b0_ref, w1_ref, b1_ref,
                    w2_ref, b2_ref, out_ref):
    z = (lax.dot_general(l1_ref[0], w0a_ref[...], (((1,), (1,)), ((), ())),
                         preferred_element_type=jnp.float32) +
         lax.dot_general(l2_ref[0], w0b_ref[...], (((1,), (1,)), ((), ())),
                         preferred_element_type=jnp.float32))
    z = jnp.maximum(z + b0_ref[...], 0.0)
    z = lax.dot_general(z, w1_ref[...], (((1,), (1,)), ((), ())),
                        preferred_element_type=jnp.float32)
    z = jnp.maximum(z + b1_ref[...], 0.0)
    out = lax.dot_general(w2_ref[...], z, (((1,), (1,)), ((), ())),
                          preferred_element_type=jnp.float32)
    out_ref[0] = out + b2_ref[...]


def _final_mlp(l1, l2, w0, b0, w1, b1, w2, b2):
    b, n, c = l1.shape
    c3 = w2.shape[0]
    w0a = w0[:, :c]
    w0b = w0[:, c:]
    full = lambda a: pl.BlockSpec(a.shape, lambda i, j: (0,) * a.ndim)
    args = (w0a, w0b, b0.reshape(1, -1), w1, b1.reshape(1, -1), w2,
            b2.reshape(-1, 1))
    return pl.pallas_call(
        _final_mlp_body,
        grid=(b, n // _NB),
        in_specs=[pl.BlockSpec((1, _NB, c), lambda i, j: (i, j, 0)),
                  pl.BlockSpec((1, _NB, c), lambda i, j: (i, j, 0))] +
                 [full(a) for a in args],
        out_specs=pl.BlockSpec((1, c3, _NB), lambda i, j: (i, 0, j)),
        out_shape=jax.ShapeDtypeStruct((b, c3, n), jnp.float32),
    )(l1, l2, *args)


def kernel(points, features, m1_w0, m1_b0, m1_w1, m1_b1, m1_w2, m1_b2,
           m2_w0, m2_b0, m2_w1, m2_b1, m2_w2, m2_b2,
           mm_w0, mm_b0, mm_w1, mm_b1, mm_w2, mm_b2):
    b, c, n = features.shape

    outs = []
    for bi in range(b):
        pts = lax.slice_in_dim(points, bi, bi + 1, axis=0)
        fts = lax.slice_in_dim(features, bi, bi + 1, axis=0)
        idx64 = _topk64(pts)
        m1 = _node_mlp(fts, m1_w0, m1_b0, m1_w1, m1_b1, m1_w2, m1_b2)
        m2 = _node_mlp(fts, m2_w0, m2_b0, m2_w1, m2_b1, m2_w2, m2_b2)
        idx1 = idx64[:, :, :KNN].reshape(n * KNN // 128, 128)
        idx2 = idx64[:, :, ::DIL].reshape(n * KNN // 128, 128)
        l1, l2 = _gather_max_sc(idx1, idx2,
                                m1.reshape(n, c), m2.reshape(n, c))
        outs.append(_final_mlp(l1.reshape(1, n, c), l2.reshape(1, n, c),
                               mm_w0, mm_b0, mm_w1, mm_b1, mm_w2, mm_b2))
    return jnp.concatenate(outs, axis=0)
